# Initial kernel scaffold; baseline (speedup 1.0000x reference)
import jax
import jax.numpy as jnp
from jax.experimental import pallas as pl


def kernel(x_sparse, f_sparse, x_dense, kW1, kb1, kW2, kb2, kW3, kb3, pW1, pb1, pW2, pb2, pW3, pb3):
    raise NotImplementedError("write your pallas kernel here")


# speedup vs baseline: 27.8759x; 27.8759x over previous
import functools

import jax
import jax.numpy as jnp
from jax.experimental import pallas as pl
from jax.experimental.pallas import tpu as pltpu

RADIUS = 0.08
BAND = 0.18
QT = 128
SC = 128
HID = 12
DIM = 3


def _gelu(x):
    return 0.5 * x * (1.0 + jax.lax.erf(x * 0.7071067811865476))


def _bf(x):
    return x.astype(jnp.bfloat16).astype(jnp.float32)


def _gno_tile_kernel(nt_s,
                     qt_zlo_ref, qt_zhi_ref, ck_zlo_ref, ck_zhi_ref,
                     kW1_ref, kb1_ref, kW2_ref, kb2_ref, kW3_ref, kb3_ref,
                     pW1_ref, pb1_ref, pW2_ref, pb2_ref, pW3_ref, pb3_ref,
                     xq_ref, xsT_ref, fsT_ref,
                     out_ref,
                     num0_ref, num1_ref, num2_ref, cnt_ref):
    i = pl.program_id(0)
    qlo = qt_zlo_ref[i] - BAND
    qhi = qt_zhi_ref[i] + BAND

    num0_ref[...] = jnp.zeros((QT, SC), jnp.float32)
    num1_ref[...] = jnp.zeros((QT, SC), jnp.float32)
    num2_ref[...] = jnp.zeros((QT, SC), jnp.float32)
    cnt_ref[...] = jnp.zeros((QT, SC), jnp.float32)

    xq0 = xq_ref[:, 0:1]
    xq1 = xq_ref[:, 1:2]
    xq2 = xq_ref[:, 2:3]
    nq = (xq0 * xq0 + xq1 * xq1) + xq2 * xq2
    qb0 = _bf(xq0)
    qb1 = _bf(xq1)
    qb2 = _bf(xq2)

    def body(c, carry):
        @pl.when((ck_zlo_ref[c] <= qhi) & (ck_zhi_ref[c] >= qlo))
        def _():
            ds = pl.ds(c * SC, SC)
            xs0 = xsT_ref[0:1, ds]
            xs1 = xsT_ref[1:2, ds]
            xs2 = xsT_ref[2:3, ds]
            ns = (xs0 * xs0 + xs1 * xs1) + xs2 * xs2
            dot = (qb0 * _bf(xs0) + qb1 * _bf(xs1)) + qb2 * _bf(xs2)
            sq = (nq + ns) - 2.0 * dot
            mf = (sq <= RADIUS * RADIUS).astype(jnp.float32)
            cnt_ref[...] += mf

            @pl.when(jnp.sum(mf) > 0.0)
            def _():
                relb = [_bf(xs0 - xq0), _bf(xs1 - xq1), _bf(xs2 - xq2)]
                h1 = [
                    _gelu((relb[0] * kW1_ref[0, k] + relb[1] * kW1_ref[1, k])
                          + relb[2] * kW1_ref[2, k] + kb1_ref[k])
                    for k in range(HID)
                ]
                h1 = [_bf(h) for h in h1]
                h2 = []
                for cc in range(HID):
                    acc = h1[0] * kW2_ref[0, cc]
                    for k in range(1, HID):
                        acc = acc + h1[k] * kW2_ref[k, cc]
                    h2.append(_gelu(acc + kb2_ref[cc]))
                h2 = [_bf(h) for h in h2]
                kern = []
                for d in range(DIM):
                    acc = h2[0] * kW3_ref[0, d]
                    for k in range(1, HID):
                        acc = acc + h2[k] * kW3_ref[k, d]
                    kern.append(acc + kb3_ref[d])
                f0 = fsT_ref[0:1, ds]
                f1 = fsT_ref[1:2, ds]
                f2 = fsT_ref[2:3, ds]
                num0_ref[...] += (kern[0] * f0) * mf
                num1_ref[...] += (kern[1] * f1) * mf
                num2_ref[...] += (kern[2] * f2) * mf
        return carry

    jax.lax.fori_loop(0, nt_s, body, 0)

    cnt = jnp.sum(cnt_ref[...], axis=1, keepdims=True)
    denom = jnp.maximum(cnt, 1.0)
    of = [_bf(jnp.sum(num0_ref[...], axis=1, keepdims=True) / denom),
          _bf(jnp.sum(num1_ref[...], axis=1, keepdims=True) / denom),
          _bf(jnp.sum(num2_ref[...], axis=1, keepdims=True) / denom)]

    hp = [
        _gelu((of[0] * pW1_ref[0, k] + of[1] * pW1_ref[1, k])
              + of[2] * pW1_ref[2, k] + pb1_ref[k])
        for k in range(HID)
    ]
    hp = [_bf(h) for h in hp]
    hp2 = []
    for cc in range(HID):
        acc = hp[0] * pW2_ref[0, cc]
        for k in range(1, HID):
            acc = acc + hp[k] * pW2_ref[k, cc]
        hp2.append(_bf(_gelu(acc + pb2_ref[cc])))
    outs = []
    for d in range(DIM):
        acc = hp2[0] * pW3_ref[0, d]
        for k in range(1, HID):
            acc = acc + hp2[k] * pW3_ref[k, d]
        outs.append(acc + pb3_ref[d])
    out_ref[...] = jnp.concatenate(outs, axis=1)


def kernel(x_sparse, f_sparse, x_dense, kW1, kb1, kW2, kb2, kW3, kb3,
           pW1, pb1, pW2, pb2, pW3, pb3):
    n_s = x_sparse.shape[0]
    n_q = x_dense.shape[0]
    nt_s = pl.cdiv(n_s, SC)
    nt_q = pl.cdiv(n_q, QT)
    n_s_pad = nt_s * SC
    n_q_pad = nt_q * QT

    sperm = jnp.argsort(x_sparse[:, 2])
    qperm = jnp.argsort(x_dense[:, 2])
    xs = x_sparse[sperm]
    fs = f_sparse[sperm]
    xq = x_dense[qperm]
    xs = jnp.concatenate(
        [xs, jnp.full((n_s_pad - n_s, DIM), 7.0, jnp.float32)], axis=0)
    fs = jnp.concatenate(
        [fs, jnp.zeros((n_s_pad - n_s, DIM), jnp.float32)], axis=0)
    xq = jnp.concatenate(
        [xq, jnp.full((n_q_pad - n_q, DIM), 3.0, jnp.float32)], axis=0)

    kW1b = _bf(kW1)
    kW2b = _bf(kW2)
    kW3b = _bf(kW3)
    pW1b = _bf(pW1)
    pW2b = _bf(pW2)
    pW3b = _bf(pW3)

    zq = xq[:, 2].reshape(nt_q, QT)
    qt_zlo = zq[:, 0]
    qt_zhi = zq[:, -1]
    zs = xs[:, 2].reshape(nt_s, SC)
    ck_zlo = zs[:, 0]
    ck_zhi = zs[:, -1]

    xsT = xs.T
    fsT = fs.T

    smem = functools.partial(pl.BlockSpec, memory_space=pltpu.SMEM)
    out_sorted = pl.pallas_call(
        functools.partial(_gno_tile_kernel, nt_s),
        grid=(nt_q,),
        in_specs=[
            smem((nt_q,), lambda i: (0,)),
            smem((nt_q,), lambda i: (0,)),
            smem((nt_s,), lambda i: (0,)),
            smem((nt_s,), lambda i: (0,)),
            smem((DIM, HID), lambda i: (0, 0)),
            smem((HID,), lambda i: (0,)),
            smem((HID, HID), lambda i: (0, 0)),
            smem((HID,), lambda i: (0,)),
            smem((HID, DIM), lambda i: (0, 0)),
            smem((DIM,), lambda i: (0,)),
            smem((DIM, HID), lambda i: (0, 0)),
            smem((HID,), lambda i: (0,)),
            smem((HID, HID), lambda i: (0, 0)),
            smem((HID,), lambda i: (0,)),
            smem((HID, DIM), lambda i: (0, 0)),
            smem((DIM,), lambda i: (0,)),
            pl.BlockSpec((QT, DIM), lambda i: (i, 0)),
            pl.BlockSpec((DIM, n_s_pad), lambda i: (0, 0)),
            pl.BlockSpec((DIM, n_s_pad), lambda i: (0, 0)),
        ],
        out_specs=pl.BlockSpec((QT, DIM), lambda i: (i, 0)),
        out_shape=jax.ShapeDtypeStruct((n_q_pad, DIM), jnp.float32),
        scratch_shapes=[pltpu.VMEM((QT, SC), jnp.float32)] * 4,
    )(qt_zlo, qt_zhi, ck_zlo, ck_zhi,
      kW1b, kb1, kW2b, kb2, kW3b, kb3, pW1b, pb1, pW2b, pb2, pW3b, pb3,
      xq, xsT, fsT)

    out = jnp.zeros((n_q, DIM), jnp.float32).at[qperm].set(out_sorted[:n_q])
    return out


# speedup vs baseline: 27.9396x; 1.0023x over previous
import functools

import jax
import jax.numpy as jnp
from jax.experimental import pallas as pl
from jax.experimental.pallas import tpu as pltpu

RADIUS = 0.08
BAND = 0.18
QT = 128
SC = 128
HID = 12
DIM = 3


def _gelu(x):
    return 0.5 * x * (1.0 + jax.lax.erf(x * 0.7071067811865476))


def _bf(x):
    return x.astype(jnp.bfloat16).astype(jnp.float32)


def _round_weights_kernel(*refs):
    n = len(refs) // 2
    for i in range(n):
        refs[n + i][...] = _bf(refs[i][...])


def _round_weights(*ws):
    return pl.pallas_call(
        _round_weights_kernel,
        out_shape=[jax.ShapeDtypeStruct(w.shape, jnp.float32) for w in ws],
    )(*ws)


def _gno_tile_kernel(nt_s,
                     qt_zlo_ref, qt_zhi_ref, ck_zlo_ref, ck_zhi_ref,
                     kW1_ref, kb1_ref, kW2_ref, kb2_ref, kW3_ref, kb3_ref,
                     pW1_ref, pb1_ref, pW2_ref, pb2_ref, pW3_ref, pb3_ref,
                     xq_ref, xsT_ref, fsT_ref,
                     out_ref,
                     num0_ref, num1_ref, num2_ref, cnt_ref):
    i = pl.program_id(0)
    qlo = qt_zlo_ref[i] - BAND
    qhi = qt_zhi_ref[i] + BAND

    num0_ref[...] = jnp.zeros((QT, SC), jnp.float32)
    num1_ref[...] = jnp.zeros((QT, SC), jnp.float32)
    num2_ref[...] = jnp.zeros((QT, SC), jnp.float32)
    cnt_ref[...] = jnp.zeros((QT, SC), jnp.float32)

    xq0 = xq_ref[:, 0:1]
    xq1 = xq_ref[:, 1:2]
    xq2 = xq_ref[:, 2:3]
    nq = (xq0 * xq0 + xq1 * xq1) + xq2 * xq2
    qb0 = _bf(xq0)
    qb1 = _bf(xq1)
    qb2 = _bf(xq2)

    def body(c, carry):
        @pl.when((ck_zlo_ref[c] <= qhi) & (ck_zhi_ref[c] >= qlo))
        def _():
            ds = pl.ds(c * SC, SC)
            xs0 = xsT_ref[0:1, ds]
            xs1 = xsT_ref[1:2, ds]
            xs2 = xsT_ref[2:3, ds]
            ns = (xs0 * xs0 + xs1 * xs1) + xs2 * xs2
            dot = (qb0 * _bf(xs0) + qb1 * _bf(xs1)) + qb2 * _bf(xs2)
            sq = (nq + ns) - 2.0 * dot
            mf = (sq <= RADIUS * RADIUS).astype(jnp.float32)
            cnt_ref[...] += mf

            @pl.when(jnp.sum(mf) > 0.0)
            def _():
                relb = [_bf(xs0 - xq0), _bf(xs1 - xq1), _bf(xs2 - xq2)]
                h1 = [
                    _gelu((relb[0] * kW1_ref[0, k] + relb[1] * kW1_ref[1, k])
                          + relb[2] * kW1_ref[2, k] + kb1_ref[k])
                    for k in range(HID)
                ]
                h1 = [_bf(h) for h in h1]
                h2 = []
                for cc in range(HID):
                    acc = h1[0] * kW2_ref[0, cc]
                    for k in range(1, HID):
                        acc = acc + h1[k] * kW2_ref[k, cc]
                    h2.append(_gelu(acc + kb2_ref[cc]))
                h2 = [_bf(h) for h in h2]
                kern = []
                for d in range(DIM):
                    acc = h2[0] * kW3_ref[0, d]
                    for k in range(1, HID):
                        acc = acc + h2[k] * kW3_ref[k, d]
                    kern.append(acc + kb3_ref[d])
                f0 = fsT_ref[0:1, ds]
                f1 = fsT_ref[1:2, ds]
                f2 = fsT_ref[2:3, ds]
                num0_ref[...] += (kern[0] * f0) * mf
                num1_ref[...] += (kern[1] * f1) * mf
                num2_ref[...] += (kern[2] * f2) * mf
        return carry

    jax.lax.fori_loop(0, nt_s, body, 0)

    cnt = jnp.sum(cnt_ref[...], axis=1, keepdims=True)
    denom = jnp.maximum(cnt, 1.0)
    of = [_bf(jnp.sum(num0_ref[...], axis=1, keepdims=True) / denom),
          _bf(jnp.sum(num1_ref[...], axis=1, keepdims=True) / denom),
          _bf(jnp.sum(num2_ref[...], axis=1, keepdims=True) / denom)]

    hp = [
        _gelu((of[0] * pW1_ref[0, k] + of[1] * pW1_ref[1, k])
              + of[2] * pW1_ref[2, k] + pb1_ref[k])
        for k in range(HID)
    ]
    hp = [_bf(h) for h in hp]
    hp2 = []
    for cc in range(HID):
        acc = hp[0] * pW2_ref[0, cc]
        for k in range(1, HID):
            acc = acc + hp[k] * pW2_ref[k, cc]
        hp2.append(_bf(_gelu(acc + pb2_ref[cc])))
    outs = []
    for d in range(DIM):
        acc = hp2[0] * pW3_ref[0, d]
        for k in range(1, HID):
            acc = acc + hp2[k] * pW3_ref[k, d]
        outs.append(acc + pb3_ref[d])
    out_ref[...] = jnp.concatenate(outs, axis=1)


def kernel(x_sparse, f_sparse, x_dense, kW1, kb1, kW2, kb2, kW3, kb3,
           pW1, pb1, pW2, pb2, pW3, pb3):
    n_s = x_sparse.shape[0]
    n_q = x_dense.shape[0]
    nt_s = pl.cdiv(n_s, SC)
    nt_q = pl.cdiv(n_q, QT)
    n_s_pad = nt_s * SC
    n_q_pad = nt_q * QT

    sperm = jnp.argsort(x_sparse[:, 2])
    qperm = jnp.argsort(x_dense[:, 2])
    xs = x_sparse[sperm]
    fs = f_sparse[sperm]
    xq = x_dense[qperm]
    xs = jnp.concatenate(
        [xs, jnp.full((n_s_pad - n_s, DIM), 7.0, jnp.float32)], axis=0)
    fs = jnp.concatenate(
        [fs, jnp.zeros((n_s_pad - n_s, DIM), jnp.float32)], axis=0)
    xq = jnp.concatenate(
        [xq, jnp.full((n_q_pad - n_q, DIM), 3.0, jnp.float32)], axis=0)

    kW1b, kW2b, kW3b, pW1b, pW2b, pW3b = _round_weights(
        kW1, kW2, kW3, pW1, pW2, pW3)

    zq = xq[:, 2].reshape(nt_q, QT)
    qt_zlo = zq[:, 0]
    qt_zhi = zq[:, -1]
    zs = xs[:, 2].reshape(nt_s, SC)
    ck_zlo = zs[:, 0]
    ck_zhi = zs[:, -1]

    xsT = xs.T
    fsT = fs.T

    smem = functools.partial(pl.BlockSpec, memory_space=pltpu.SMEM)
    out_sorted = pl.pallas_call(
        functools.partial(_gno_tile_kernel, nt_s),
        grid=(nt_q,),
        in_specs=[
            smem((nt_q,), lambda i: (0,)),
            smem((nt_q,), lambda i: (0,)),
            smem((nt_s,), lambda i: (0,)),
            smem((nt_s,), lambda i: (0,)),
            smem((DIM, HID), lambda i: (0, 0)),
            smem((HID,), lambda i: (0,)),
            smem((HID, HID), lambda i: (0, 0)),
            smem((HID,), lambda i: (0,)),
            smem((HID, DIM), lambda i: (0, 0)),
            smem((DIM,), lambda i: (0,)),
            smem((DIM, HID), lambda i: (0, 0)),
            smem((HID,), lambda i: (0,)),
            smem((HID, HID), lambda i: (0, 0)),
            smem((HID,), lambda i: (0,)),
            smem((HID, DIM), lambda i: (0, 0)),
            smem((DIM,), lambda i: (0,)),
            pl.BlockSpec((QT, DIM), lambda i: (i, 0)),
            pl.BlockSpec((DIM, n_s_pad), lambda i: (0, 0)),
            pl.BlockSpec((DIM, n_s_pad), lambda i: (0, 0)),
        ],
        out_specs=pl.BlockSpec((QT, DIM), lambda i: (i, 0)),
        out_shape=jax.ShapeDtypeStruct((n_q_pad, DIM), jnp.float32),
        scratch_shapes=[pltpu.VMEM((QT, SC), jnp.float32)] * 4,
    )(qt_zlo, qt_zhi, ck_zlo, ck_zhi,
      kW1b, kb1, kW2b, kb2, kW3b, kb3, pW1b, pb1, pW2b, pb2, pW3b, pb3,
      xq, xsT, fsT)

    out = jnp.zeros((n_q, DIM), jnp.float32).at[qperm].set(out_sorted[:n_q])
    return out


# speedup vs baseline: 41.7232x; 1.4933x over previous
import functools

import jax
import jax.numpy as jnp
from jax.experimental import pallas as pl
from jax.experimental.pallas import tpu as pltpu

RADIUS = 0.08
BAND = 0.18
NBINS = 9
QT = 128
SC = 128
HID = 12
DIM = 3


def _gelu(x):
    return 0.5 * x * (1.0 + jax.lax.erf(x * 0.7071067811865476))


def _bf(x):
    return x.astype(jnp.bfloat16).astype(jnp.float32)


def _round_weights_kernel(*refs):
    n = len(refs) // 2
    for i in range(n):
        refs[n + i][...] = _bf(refs[i][...])


def _round_weights(*ws):
    return pl.pallas_call(
        _round_weights_kernel,
        out_shape=[jax.ShapeDtypeStruct(w.shape, jnp.float32) for w in ws],
    )(*ws)


def _gno_tile_kernel(nt_s,
                     qt_zlo_ref, qt_zhi_ref, ck_zlo_ref, ck_zhi_ref,
                     qt_ylo_ref, qt_yhi_ref, ck_ylo_ref, ck_yhi_ref,
                     kW1_ref, kb1_ref, kW2_ref, kb2_ref, kW3_ref, kb3_ref,
                     pW1_ref, pb1_ref, pW2_ref, pb2_ref, pW3_ref, pb3_ref,
                     xq_ref, xsT_ref, fsT_ref,
                     out_ref,
                     num0_ref, num1_ref, num2_ref, cnt_ref):
    i = pl.program_id(0)
    qlo = qt_zlo_ref[i] - BAND
    qhi = qt_zhi_ref[i] + BAND
    qylo = qt_ylo_ref[i] - BAND
    qyhi = qt_yhi_ref[i] + BAND

    num0_ref[...] = jnp.zeros((QT, SC), jnp.float32)
    num1_ref[...] = jnp.zeros((QT, SC), jnp.float32)
    num2_ref[...] = jnp.zeros((QT, SC), jnp.float32)
    cnt_ref[...] = jnp.zeros((QT, SC), jnp.float32)

    xq0 = xq_ref[:, 0:1]
    xq1 = xq_ref[:, 1:2]
    xq2 = xq_ref[:, 2:3]
    nq = (xq0 * xq0 + xq1 * xq1) + xq2 * xq2
    qb0 = _bf(xq0)
    qb1 = _bf(xq1)
    qb2 = _bf(xq2)

    def body(c, carry):
        @pl.when((ck_zlo_ref[c] <= qhi) & (ck_zhi_ref[c] >= qlo)
                 & (ck_ylo_ref[c] <= qyhi) & (ck_yhi_ref[c] >= qylo))
        def _():
            ds = pl.ds(c * SC, SC)
            xs0 = xsT_ref[0:1, ds]
            xs1 = xsT_ref[1:2, ds]
            xs2 = xsT_ref[2:3, ds]
            ns = (xs0 * xs0 + xs1 * xs1) + xs2 * xs2
            dot = (qb0 * _bf(xs0) + qb1 * _bf(xs1)) + qb2 * _bf(xs2)
            sq = (nq + ns) - 2.0 * dot
            mf = (sq <= RADIUS * RADIUS).astype(jnp.float32)
            cnt_ref[...] += mf

            @pl.when(jnp.sum(mf) > 0.0)
            def _():
                relb = [_bf(xs0 - xq0), _bf(xs1 - xq1), _bf(xs2 - xq2)]
                h1 = [
                    _gelu((relb[0] * kW1_ref[0, k] + relb[1] * kW1_ref[1, k])
                          + relb[2] * kW1_ref[2, k] + kb1_ref[k])
                    for k in range(HID)
                ]
                h1 = [_bf(h) for h in h1]
                h2 = []
                for cc in range(HID):
                    acc = h1[0] * kW2_ref[0, cc]
                    for k in range(1, HID):
                        acc = acc + h1[k] * kW2_ref[k, cc]
                    h2.append(_gelu(acc + kb2_ref[cc]))
                h2 = [_bf(h) for h in h2]
                kern = []
                for d in range(DIM):
                    acc = h2[0] * kW3_ref[0, d]
                    for k in range(1, HID):
                        acc = acc + h2[k] * kW3_ref[k, d]
                    kern.append(acc + kb3_ref[d])
                f0 = fsT_ref[0:1, ds]
                f1 = fsT_ref[1:2, ds]
                f2 = fsT_ref[2:3, ds]
                num0_ref[...] += (kern[0] * f0) * mf
                num1_ref[...] += (kern[1] * f1) * mf
                num2_ref[...] += (kern[2] * f2) * mf
        return carry

    jax.lax.fori_loop(0, nt_s, body, 0)

    cnt = jnp.sum(cnt_ref[...], axis=1, keepdims=True)
    denom = jnp.maximum(cnt, 1.0)
    of = [_bf(jnp.sum(num0_ref[...], axis=1, keepdims=True) / denom),
          _bf(jnp.sum(num1_ref[...], axis=1, keepdims=True) / denom),
          _bf(jnp.sum(num2_ref[...], axis=1, keepdims=True) / denom)]

    hp = [
        _gelu((of[0] * pW1_ref[0, k] + of[1] * pW1_ref[1, k])
              + of[2] * pW1_ref[2, k] + pb1_ref[k])
        for k in range(HID)
    ]
    hp = [_bf(h) for h in hp]
    hp2 = []
    for cc in range(HID):
        acc = hp[0] * pW2_ref[0, cc]
        for k in range(1, HID):
            acc = acc + hp[k] * pW2_ref[k, cc]
        hp2.append(_bf(_gelu(acc + pb2_ref[cc])))
    outs = []
    for d in range(DIM):
        acc = hp2[0] * pW3_ref[0, d]
        for k in range(1, HID):
            acc = acc + hp2[k] * pW3_ref[k, d]
        outs.append(acc + pb3_ref[d])
    out_ref[...] = jnp.concatenate(outs, axis=1)


def kernel(x_sparse, f_sparse, x_dense, kW1, kb1, kW2, kb2, kW3, kb3,
           pW1, pb1, pW2, pb2, pW3, pb3):
    n_s = x_sparse.shape[0]
    n_q = x_dense.shape[0]
    nt_s = pl.cdiv(n_s, SC)
    nt_q = pl.cdiv(n_q, QT)
    n_s_pad = nt_s * SC
    n_q_pad = nt_q * QT

    skey = jnp.floor(x_sparse[:, 1] * NBINS) * 4.0 + x_sparse[:, 2]
    qkey = jnp.floor(x_dense[:, 1] * NBINS) * 4.0 + x_dense[:, 2]
    sperm = jnp.argsort(skey)
    qperm = jnp.argsort(qkey)
    xs = x_sparse[sperm]
    fs = f_sparse[sperm]
    xq = x_dense[qperm]
    xs = jnp.concatenate(
        [xs, jnp.full((n_s_pad - n_s, DIM), 7.0, jnp.float32)], axis=0)
    fs = jnp.concatenate(
        [fs, jnp.zeros((n_s_pad - n_s, DIM), jnp.float32)], axis=0)
    xq = jnp.concatenate(
        [xq, jnp.full((n_q_pad - n_q, DIM), 3.0, jnp.float32)], axis=0)

    kW1b, kW2b, kW3b, pW1b, pW2b, pW3b = _round_weights(
        kW1, kW2, kW3, pW1, pW2, pW3)

    zq = xq[:, 2].reshape(nt_q, QT)
    qt_zlo = zq.min(axis=1)
    qt_zhi = zq.max(axis=1)
    zs = xs[:, 2].reshape(nt_s, SC)
    ck_zlo = zs.min(axis=1)
    ck_zhi = zs.max(axis=1)
    yq = xq[:, 1].reshape(nt_q, QT)
    qt_ylo = yq.min(axis=1)
    qt_yhi = yq.max(axis=1)
    ys = xs[:, 1].reshape(nt_s, SC)
    ck_ylo = ys.min(axis=1)
    ck_yhi = ys.max(axis=1)

    xsT = xs.T
    fsT = fs.T

    smem = functools.partial(pl.BlockSpec, memory_space=pltpu.SMEM)
    out_sorted = pl.pallas_call(
        functools.partial(_gno_tile_kernel, nt_s),
        grid=(nt_q,),
        in_specs=[
            smem((nt_q,), lambda i: (0,)),
            smem((nt_q,), lambda i: (0,)),
            smem((nt_s,), lambda i: (0,)),
            smem((nt_s,), lambda i: (0,)),
            smem((nt_q,), lambda i: (0,)),
            smem((nt_q,), lambda i: (0,)),
            smem((nt_s,), lambda i: (0,)),
            smem((nt_s,), lambda i: (0,)),
            smem((DIM, HID), lambda i: (0, 0)),
            smem((HID,), lambda i: (0,)),
            smem((HID, HID), lambda i: (0, 0)),
            smem((HID,), lambda i: (0,)),
            smem((HID, DIM), lambda i: (0, 0)),
            smem((DIM,), lambda i: (0,)),
            smem((DIM, HID), lambda i: (0, 0)),
            smem((HID,), lambda i: (0,)),
            smem((HID, HID), lambda i: (0, 0)),
            smem((HID,), lambda i: (0,)),
            smem((HID, DIM), lambda i: (0, 0)),
            smem((DIM,), lambda i: (0,)),
            pl.BlockSpec((QT, DIM), lambda i: (i, 0)),
            pl.BlockSpec((DIM, n_s_pad), lambda i: (0, 0)),
            pl.BlockSpec((DIM, n_s_pad), lambda i: (0, 0)),
        ],
        out_specs=pl.BlockSpec((QT, DIM), lambda i: (i, 0)),
        out_shape=jax.ShapeDtypeStruct((n_q_pad, DIM), jnp.float32),
        scratch_shapes=[pltpu.VMEM((QT, SC), jnp.float32)] * 4,
    )(qt_zlo, qt_zhi, ck_zlo, ck_zhi,
      qt_ylo, qt_yhi, ck_ylo, ck_yhi,
      kW1b, kb1, kW2b, kb2, kW3b, kb3, pW1b, pb1, pW2b, pb2, pW3b, pb3,
      xq, xsT, fsT)

    out = jnp.zeros((n_q, DIM), jnp.float32).at[qperm].set(out_sorted[:n_q])
    return out
